# baseline (speedup 1.0000x reference)
import functools

import jax
import jax.numpy as jnp
from jax import lax
from jax.experimental import pallas as pl
from jax.experimental.pallas import tpu as pltpu
from jax.experimental.pallas import tpu_sc as plsc

VOCAB = 50000
DIM = 64
NUM_HEADS = 4
B, L = 4, 8192
TOK = B * L

_M1 = (10007, 10009, 10037, 10039)
_M2 = (20011, 20021, 20023, 20029)
_C1 = tuple((256 * m) % VOCAB for m in _M1)
_C2 = tuple((256 * m) % VOCAB for m in _M2)

_LANES = 16
_NW = 32
_TPW = TOK // _NW
_CHUNK = 256
_NCHUNK = _TPW // _CHUNK
_STREAM = 128
_NHALF = _CHUNK // _STREAM


def _lt(v, idx):
    return v.at[idx].get(mode="promise_in_bounds")


def _mod_v(y):
    q = (y.astype(jnp.float32) * (1.0 / VOCAB)).astype(jnp.int32)
    r = y - q * VOCAB
    r = jnp.where(r < 0, r + VOCAB, r)
    return jnp.where(r >= VOCAB, r - VOCAB, r)


@functools.partial(
    pl.kernel,
    mesh=plsc.VectorSubcoreMesh(core_axis_name="c", subcore_axis_name="s"),
    out_type=jax.ShapeDtypeStruct((B, L, NUM_HEADS * DIM), jnp.float32),
    scratch_types=[
        pltpu.VMEM((_CHUNK,), jnp.int32),
        pltpu.VMEM((_LANES,), jnp.int32),
        pltpu.VMEM((NUM_HEADS * _CHUNK,), jnp.int32),
        pltpu.VMEM((NUM_HEADS, _CHUNK, DIM), jnp.float32),
        pltpu.SemaphoreType.DMA,
    ],
    compiler_params=pltpu.CompilerParams(use_tc_tiling_on_sc=False),
)
def _engram_sc(ids_ref, table_ref, out_ref, win, pre, idxb, slab, sem):
    wid = lax.axis_index("s") * 2 + lax.axis_index("c")
    row = lax.shift_right_logical(wid, 3)
    col0 = (wid & 7) * _TPW
    iota = lax.iota(jnp.int32, _LANES)
    shift_idx = jnp.maximum(iota - 1, 0)
    last_idx = iota * 0 + (_LANES - 1)

    for c in range(_NCHUNK):
        col = col0 + c * _CHUNK
        pltpu.sync_copy(
            ids_ref.at[row, pl.ds(pl.multiple_of(col, 8), _CHUNK)], win
        )
        pltpu.sync_copy(
            ids_ref.at[
                row,
                pl.ds(pl.multiple_of(jnp.maximum(col - _LANES, 0), 8), _LANES),
            ],
            pre,
        )
        carry = _lt(pre[...], last_idx)

        for i in range(_CHUNK // _LANES):
            ids = win[pl.ds(i * _LANES, _LANES)]
            prev = jnp.where(iota == 0, carry, _lt(ids, shift_idx))
            carry = _lt(ids, last_idx)
            tokpos = iota + (col + i * _LANES)
            prev = jnp.where((tokpos & (L - 1)) == 0, 0, prev)
            id_hi = lax.shift_right_logical(ids, 8)
            id_lo = ids & 255
            pv_hi = lax.shift_right_logical(prev, 8)
            pv_lo = prev & 255
            half = i // (_STREAM // _LANES)
            off = (i % (_STREAM // _LANES)) * _LANES
            for k in range(NUM_HEADS):
                h = _mod_v(id_hi * _C1[k] + id_lo * _M1[k]) + _mod_v(
                    pv_hi * _C2[k] + pv_lo * _M2[k]
                )
                h = jnp.where(h >= VOCAB, h - VOCAB, h)
                idxb[pl.ds((k * _NHALF + half) * _STREAM + off, _LANES)] = h

        copies = []
        for k in range(NUM_HEADS):
            for half in range(_NHALF):
                copies.append(
                    pltpu.async_copy(
                        table_ref.at[idxb.at[pl.ds((k * _NHALF + half) * _STREAM, _STREAM)]],
                        slab.at[k, pl.ds(half * _STREAM, _STREAM)],
                        sem,
                    )
                )
        for cp in copies:
            cp.wait()

        for k in range(NUM_HEADS):
            pltpu.sync_copy(
                slab.at[k],
                out_ref.at[
                    row,
                    pl.ds(pl.multiple_of(col, 8), _CHUNK),
                    pl.ds(k * DIM, DIM),
                ],
            )


def kernel(hidden_states, input_ids, table):
    del hidden_states
    return _engram_sc(input_ids, table)
